# baseline (speedup 1.0000x reference)
import functools

import jax
import jax.numpy as jnp
import numpy as np
from jax import lax
from jax.experimental import pallas as pl
from jax.experimental.pallas import tpu as pltpu
from jax.experimental.pallas import tpu_sc as plsc

F = 26
V = 3846
S = 99996
D = 16
B = 4096
NT = 32
RPT = B // NT
G = 32
FS = F * S
SP = 100000
FR = FS + SP
IDXW = (F + 1) * G


def _sc_ffm(flat16, xoffT):
    mesh = plsc.VectorSubcoreMesh(core_axis_name="c", subcore_axis_name="s")

    @functools.partial(
        pl.kernel,
        out_type=jax.ShapeDtypeStruct((B * D,), jnp.float32),
        mesh=mesh,
        scratch_types=[
            pltpu.VMEM((G, RPT), jnp.int32),
            pltpu.VMEM((IDXW,), jnp.int32),
            pltpu.VMEM((IDXW, D), jnp.float32),
            pltpu.VMEM((RPT * D,), jnp.float32),
            pltpu.SemaphoreType.DMA,
        ],
        compiler_params=pltpu.CompilerParams(
            use_tc_tiling_on_sc=False, needs_layout_passes=False),
    )
    def kern(tab_hbm, xo_hbm, z_hbm, xoff_v, idx_v, gbuf, zloc, sem):
        wid = lax.axis_index("s") * 2 + lax.axis_index("c")
        base = wid * RPT
        pltpu.sync_copy(xo_hbm.at[:, pl.ds(base, RPT)], xoff_v)

        lanes = lax.iota(jnp.int32, 16)

        @pl.loop(0, RPT)
        def _(r):
            rv = jnp.full((16,), r, jnp.int32)
            xv0 = plsc.load_gather(xoff_v, [lanes, rv])
            xv1 = plsc.load_gather(xoff_v, [lanes + D, rv])
            xc1 = jnp.minimum(xv1, S - 1)
            for t in range(F):
                idx_v[pl.ds(t * G, D)] = xv0 + t * S
                idx_v[pl.ds(t * G + D, D)] = xc1 + t * S
            idx_v[pl.ds(F * G, D)] = xv0 + FS
            idx_v[pl.ds(F * G + D, D)] = xv1 + FS
            copies = []
            for c in range(IDXW // 128):
                sl = pl.ds(c * 128, 128)
                copies.append(
                    pltpu.async_copy(tab_hbm.at[idx_v.at[sl]], gbuf.at[sl],
                                     sem))
            rem = IDXW % 128
            if rem:
                sl = pl.ds(IDXW - rem, rem)
                copies.append(
                    pltpu.async_copy(tab_hbm.at[idx_v.at[sl]], gbuf.at[sl],
                                     sem))
            for cp in copies:
                cp.wait()

            acc = gbuf[F * G]
            for f in range(1, F):
                acc = acc + gbuf[F * G + f]
            for i in range(F - 1):
                for j in range(i + 1, F):
                    acc = acc + gbuf[i * G + j] * gbuf[j * G + i]
            zloc[pl.ds(r * D, D)] = acc

        pltpu.sync_copy(zloc, z_hbm.at[pl.ds(base * D, RPT * D)])

    return kern(flat16, xoffT)


def _tc_finish(z2d, bias):
    def body(z_ref, b_ref, o_ref):
        o_ref[...] = jax.nn.sigmoid(jnp.sum(z_ref[...], axis=1) + b_ref[0])

    return pl.pallas_call(
        body,
        out_shape=jax.ShapeDtypeStruct((B,), jnp.float32),
    )(z2d, bias)


@jax.jit
def kernel(x, fc_weight, bias, ffm_tables):
    offsets = np.arange(F, dtype=np.int32) * V
    x_off = x.astype(jnp.int32) + jnp.asarray(offsets)[None, :]
    xoffT = jnp.concatenate(
        [x_off.T, jnp.full((G - F, B), S, jnp.int32)], axis=0)

    tab16 = ffm_tables.reshape(FS, D)
    fc16 = jnp.pad(fc_weight, ((0, SP - S), (0, D - 1)))
    flat16 = jnp.concatenate([tab16, fc16], axis=0)

    z = _sc_ffm(flat16, xoffT)
    return _tc_finish(z.reshape(B, D), bias)


# speedup vs baseline: 1.8409x; 1.8409x over previous
import functools

import jax
import jax.numpy as jnp
import numpy as np
from jax import lax
from jax.experimental import pallas as pl
from jax.experimental.pallas import tpu as pltpu
from jax.experimental.pallas import tpu_sc as plsc

F = 26
V = 3846
S = 99996
D = 16
B = 4096
NT = 32
RPT = B // NT
G = 32
FS = F * S
SP = 100000
FR = FS + SP
IDXW = (F + 1) * G


def _sc_linearize(tab128, fc128):
    mesh = plsc.VectorSubcoreMesh(core_axis_name="c", subcore_axis_name="s")
    TW = FS * D // 128
    FW = SP * D // 128
    TN = TW // NT
    FN = FW // NT
    CN = 256

    @functools.partial(
        pl.kernel,
        out_type=jax.ShapeDtypeStruct((FR, D), jnp.float32),
        mesh=mesh,
        scratch_types=[
            pltpu.VMEM((CN, 128), jnp.float32),
            pltpu.VMEM((CN * 8, D), jnp.float32),
        ],
        compiler_params=pltpu.CompilerParams(
            use_tc_tiling_on_sc=False, needs_layout_passes=False),
    )
    def kern(t_hbm, f_hbm, o_hbm, buf, b16):
        wid = lax.axis_index("s") * 2 + lax.axis_index("c")

        def shuffle(n):
            @pl.loop(0, n)
            def _(q):
                for h in range(8):
                    b16[q * 8 + h, :] = buf[q, pl.ds(16 * h, D)]

        nfull = TN // CN
        @pl.loop(0, nfull)
        def _(c):
            a = wid * TN + c * CN
            pltpu.sync_copy(t_hbm.at[pl.ds(a, CN), :], buf)
            shuffle(CN)
            pltpu.sync_copy(b16, o_hbm.at[pl.ds(a * 8, CN * 8), :])

        rem = TN % CN
        a = wid * TN + nfull * CN
        pltpu.sync_copy(t_hbm.at[pl.ds(a, rem), :], buf.at[pl.ds(0, rem), :])
        shuffle(rem)
        pltpu.sync_copy(b16.at[pl.ds(0, rem * 8), :],
                        o_hbm.at[pl.ds(a * 8, rem * 8), :])

        fa = wid * FN
        pltpu.sync_copy(f_hbm.at[pl.ds(fa, FN), :], buf.at[pl.ds(0, FN), :])
        shuffle(FN)
        pltpu.sync_copy(b16.at[pl.ds(0, FN * 8), :],
                        o_hbm.at[pl.ds(FS + fa * 8, FN * 8), :])

        @pl.when(wid == NT - 1)
        def _():
            r1 = TW - TN * NT
            pltpu.sync_copy(t_hbm.at[pl.ds(TN * NT, r1), :],
                            buf.at[pl.ds(0, r1), :])
            shuffle(r1)
            pltpu.sync_copy(b16.at[pl.ds(0, r1 * 8), :],
                            o_hbm.at[pl.ds(TN * NT * 8, r1 * 8), :])
            r2 = FW - FN * NT
            pltpu.sync_copy(f_hbm.at[pl.ds(FN * NT, r2), :],
                            buf.at[pl.ds(0, r2), :])
            shuffle(r2)
            pltpu.sync_copy(b16.at[pl.ds(0, r2 * 8), :],
                            o_hbm.at[pl.ds(FS + FN * NT * 8, r2 * 8), :])

    return kern(tab128, fc128)


def _sc_ffm(flat16, xoffT):
    mesh = plsc.VectorSubcoreMesh(core_axis_name="c", subcore_axis_name="s")

    @functools.partial(
        pl.kernel,
        out_type=jax.ShapeDtypeStruct((B * D,), jnp.float32),
        mesh=mesh,
        scratch_types=[
            pltpu.VMEM((G, RPT), jnp.int32),
            pltpu.VMEM((IDXW,), jnp.int32),
            pltpu.VMEM((IDXW, D), jnp.float32),
            pltpu.VMEM((RPT * D,), jnp.float32),
            pltpu.SemaphoreType.DMA,
        ],
        compiler_params=pltpu.CompilerParams(
            use_tc_tiling_on_sc=False, needs_layout_passes=False),
    )
    def kern(tab_hbm, xo_hbm, z_hbm, xoff_v, idx_v, gbuf, zloc, sem):
        wid = lax.axis_index("s") * 2 + lax.axis_index("c")
        base = wid * RPT
        pltpu.sync_copy(xo_hbm.at[:, pl.ds(base, RPT)], xoff_v)

        lanes = lax.iota(jnp.int32, 16)

        @pl.loop(0, RPT)
        def _(r):
            rv = jnp.full((16,), r, jnp.int32)
            xv0 = plsc.load_gather(xoff_v, [lanes, rv])
            xv1 = plsc.load_gather(xoff_v, [lanes + D, rv])
            xc1 = jnp.minimum(xv1, S - 1)
            for t in range(F):
                idx_v[pl.ds(t * G, D)] = xv0 + t * S
                idx_v[pl.ds(t * G + D, D)] = xc1 + t * S
            idx_v[pl.ds(F * G, D)] = xv0 + FS
            idx_v[pl.ds(F * G + D, D)] = xv1 + FS
            copies = []
            for c in range(IDXW // 128):
                sl = pl.ds(c * 128, 128)
                copies.append(
                    pltpu.async_copy(tab_hbm.at[idx_v.at[sl]], gbuf.at[sl],
                                     sem))
            rem = IDXW % 128
            if rem:
                sl = pl.ds(IDXW - rem, rem)
                copies.append(
                    pltpu.async_copy(tab_hbm.at[idx_v.at[sl]], gbuf.at[sl],
                                     sem))
            for cp in copies:
                cp.wait()

            acc = gbuf[F * G]
            for f in range(1, F):
                acc = acc + gbuf[F * G + f]
            for i in range(F - 1):
                for j in range(i + 1, F):
                    acc = acc + gbuf[i * G + j] * gbuf[j * G + i]
            zloc[pl.ds(r * D, D)] = acc

        pltpu.sync_copy(zloc, z_hbm.at[pl.ds(base * D, RPT * D)])

    return kern(flat16, xoffT)


def _tc_finish(z2d, bias):
    def body(z_ref, b_ref, o_ref):
        o_ref[...] = jax.nn.sigmoid(jnp.sum(z_ref[...], axis=1) + b_ref[0])

    return pl.pallas_call(
        body,
        out_shape=jax.ShapeDtypeStruct((B,), jnp.float32),
    )(z2d, bias)


@jax.jit
def kernel(x, fc_weight, bias, ffm_tables):
    offsets = np.arange(F, dtype=np.int32) * V
    x_off = x.astype(jnp.int32) + jnp.asarray(offsets)[None, :]
    xoffT = jnp.concatenate(
        [x_off.T, jnp.full((G - F, B), S, jnp.int32)], axis=0)

    tab128 = ffm_tables.reshape(FS * D // 128, 128)
    fc16 = jnp.pad(fc_weight, ((0, SP - S), (0, D - 1)))
    fc128 = fc16.reshape(SP * D // 128, 128)
    flat16 = _sc_linearize(tab128, fc128)

    z = _sc_ffm(flat16, xoffT)
    return _tc_finish(z.reshape(B, D), bias)


# speedup vs baseline: 5.4054x; 2.9364x over previous
import functools

import jax
import jax.numpy as jnp
import numpy as np
from jax import lax
from jax.experimental import pallas as pl
from jax.experimental.pallas import tpu as pltpu
from jax.experimental.pallas import tpu_sc as plsc

F = 26
V = 3846
S = 99996
D = 16
B = 4096
NT = 32
RPT = B // NT
G = 32
NP = 4
FC = 26
NBUF = 4


def _sc_ffm(t0, t1, t2, t3, xoffT):
    mesh = plsc.VectorSubcoreMesh(core_axis_name="c", subcore_axis_name="s")

    @functools.partial(
        pl.kernel,
        out_type=jax.ShapeDtypeStruct((B * D,), jnp.float32),
        mesh=mesh,
        scratch_types=[
            pltpu.VMEM((G, RPT), jnp.int32),
            pltpu.VMEM((G,), jnp.int32),
            pltpu.VMEM((NP, G, 128), jnp.float32),
            pltpu.VMEM((RPT * D,), jnp.float32),
            pltpu.SemaphoreType.DMA,
        ],
        compiler_params=pltpu.CompilerParams(
            use_tc_tiling_on_sc=False, needs_layout_passes=False),
    )
    def kern(t0_hbm, t1_hbm, t2_hbm, t3_hbm, xo_hbm, z_hbm,
             xoff_v, idx_v, gbuf, zloc, sem):
        wid = lax.axis_index("s") * 2 + lax.axis_index("c")
        base = wid * RPT
        pltpu.sync_copy(xo_hbm.at[:, pl.ds(base, RPT)], xoff_v)
        tabs = (t0_hbm, t1_hbm, t2_hbm, t3_hbm)

        lanes = lax.iota(jnp.int32, 16)

        @pl.loop(0, RPT)
        def _(r):
            rv = jnp.full((16,), r, jnp.int32)
            xv0 = plsc.load_gather(xoff_v, [lanes, rv])
            xv1 = plsc.load_gather(xoff_v, [lanes + D, rv])
            idx_v[pl.ds(0, D)] = xv0
            idx_v[pl.ds(D, D)] = jnp.minimum(xv1, S - 1)
            copies = [
                pltpu.async_copy(tabs[p].at[idx_v], gbuf.at[p], sem)
                for p in range(NP)
            ]
            for cp in copies:
                cp.wait()

            acc = gbuf[FC // 8, 0, pl.ds((FC % 8) * D, D)]
            for f in range(1, F):
                acc = acc + gbuf[FC // 8, f, pl.ds((FC % 8) * D, D)]
            for i in range(F - 1):
                for j in range(i + 1, F):
                    va = gbuf[j // 8, i, pl.ds((j % 8) * D, D)]
                    vb = gbuf[i // 8, j, pl.ds((i % 8) * D, D)]
                    acc = acc + va * vb
            zloc[pl.ds(r * D, D)] = acc

        pltpu.sync_copy(zloc, z_hbm.at[pl.ds(base * D, RPT * D)])

    return kern(t0, t1, t2, t3, xoffT)


def _tc_finish(z2d, bias):
    def body(z_ref, b_ref, o_ref):
        o_ref[...] = jax.nn.sigmoid(jnp.sum(z_ref[...], axis=1) + b_ref[0])

    return pl.pallas_call(
        body,
        out_shape=jax.ShapeDtypeStruct((B,), jnp.float32),
    )(z2d, bias)


@jax.jit
def kernel(x, fc_weight, bias, ffm_tables):
    offsets = np.arange(F, dtype=np.int32) * V
    x_off = x.astype(jnp.int32) + jnp.asarray(offsets)[None, :]
    xoffT = jnp.concatenate(
        [x_off.T, jnp.full((G - F, B), S, jnp.int32)], axis=0)

    packs = []
    for i in range(3):
        packs.append(
            ffm_tables[8 * i:8 * i + 8].transpose(1, 0, 2).reshape(S, 128))
    fc16 = jnp.concatenate([fc_weight, jnp.zeros((S, D - 1), jnp.float32)], 1)
    last = jnp.concatenate([ffm_tables[24:26], fc16[None]], axis=0)
    t3 = jnp.pad(last.transpose(1, 0, 2).reshape(S, 48), ((0, 0), (0, 80)))
    packs.append(t3)

    z = _sc_ffm(*packs, xoffT)
    return _tc_finish(z.reshape(B, D), bias)
